# baseline (speedup 1.0000x reference)
import functools

import jax
import jax.numpy as jnp
from jax import lax
from jax.experimental import pallas as pl
from jax.experimental.pallas import tpu as pltpu

BLK = 256


def _expert_chain(x, wrefs, b):
    proj = len(wrefs) == 9
    h = jnp.tanh(jnp.dot(x, wrefs[0][...]) + b[0])
    y = jnp.dot(h, wrefs[1][...]) + b[1]
    res = jnp.dot(x, wrefs[2][...]) + b[2] if proj else x
    x = jnp.tanh(res + y)
    off = 3 if proj else 2
    for k in range(2):
        wa, wb = wrefs[off + 2 * k], wrefs[off + 2 * k + 1]
        h = jnp.tanh(jnp.dot(x, wa[...]) + b[off + 2 * k])
        x = jnp.tanh(x + jnp.dot(h, wb[...]) + b[off + 2 * k + 1])
    wa, wb = wrefs[off + 4], wrefs[off + 5]
    h = jnp.tanh(jnp.dot(x, wa[...]) + b[off + 4])
    return x + jnp.dot(h, wb[...]) + b[off + 5]


def _expert_body(nt_ref, x_ref, *rest):
    *wrefs, b_ref, o_ref = rest
    i = pl.program_id(0)

    @pl.when(i < nt_ref[0])
    def _():
        o_ref[...] = _expert_chain(x_ref[...], list(wrefs), b_ref[...])


def _run_expert(x, ws, bs, nt):
    cap, din = x.shape
    grid = cap // BLK
    nmat = len(ws)
    bstack = jnp.stack(bs)
    in_specs = [pl.BlockSpec((BLK, din), lambda i, s: (i, 0))]
    for w in ws:
        in_specs.append(
            pl.BlockSpec(w.shape, lambda i, s: (0,) * w.ndim))
    in_specs.append(pl.BlockSpec((nmat, 1024), lambda i, s: (0, 0)))
    return pl.pallas_call(
        _expert_body,
        grid_spec=pltpu.PrefetchScalarGridSpec(
            num_scalar_prefetch=1,
            grid=(grid,),
            in_specs=in_specs,
            out_specs=pl.BlockSpec((BLK, 1024), lambda i, s: (i, 0)),
        ),
        out_shape=jax.ShapeDtypeStruct((cap, 1024), jnp.float32),
    )(nt, x, *ws, bstack)


def _enc_weights(p, proj):
    w = [p["first"]["W1"], p["first"]["W2"]]
    b = [p["first"]["b1"], p["first"]["b2"]]
    if proj:
        w.append(p["first"]["Wp"])
        b.append(p["first"]["bp"])
    for sp in p["stack"]:
        w += [sp["W1"], sp["W2"]]
        b += [sp["b1"], sp["b2"]]
    w += [p["last"]["W1"], p["last"]["W2"]]
    b += [p["last"]["b1"], p["last"]["b2"]]
    return w, b


def kernel(action_type, AGV_idx, op_from_idx, op_to_idx, machine_idx,
           AGV_emb, operation_emb, machine_emb, wait_emb,
           pick_params, transport_params, move_params):
    N = action_type.shape[0]
    C = AGV_emb.shape[1]
    CAP = N + BLK
    DUMP = CAP - 1

    t = action_type.astype(jnp.int32)
    is1 = (t == 1).astype(jnp.int32)
    is2 = (t == 2).astype(jnp.int32)
    is3 = (t == 3).astype(jnp.int32)
    r1 = jnp.cumsum(is1) - is1
    r2 = jnp.cumsum(is2) - is2
    r3 = jnp.cumsum(is3) - is3
    c1, c2, c3 = jnp.sum(is1), jnp.sum(is2), jnp.sum(is3)

    pos1 = jnp.where(is1 == 1, r1, DUMP)
    pos2 = jnp.where(is2 == 1, r2, DUMP)
    pos3 = jnp.where(is3 == 1, r3, DUMP)

    a = jnp.take(AGV_emb, AGV_idx, axis=0)
    m = jnp.take(machine_emb, machine_idx, axis=0)
    of = jnp.take(operation_emb, op_from_idx, axis=0)
    ot = jnp.take(operation_emb, op_to_idx, axis=0)
    pick_feat = jnp.concatenate([a, of, ot, m], axis=-1)
    tm_feat = jnp.concatenate([a, m], axis=-1)

    Xp = jnp.zeros((CAP, 4 * C), jnp.float32).at[pos1].set(pick_feat)
    Xt = jnp.zeros((CAP, 2 * C), jnp.float32).at[pos2].set(tm_feat)
    Xm = jnp.zeros((CAP, 2 * C), jnp.float32).at[pos3].set(tm_feat)

    def ntiles(c):
        return ((c + BLK - 1) // BLK).astype(jnp.int32).reshape(1)

    wp, bp = _enc_weights(pick_params, proj=False)
    wt, bt = _enc_weights(transport_params, proj=True)
    wm, bm = _enc_weights(move_params, proj=True)

    Yp = _run_expert(Xp, wp, bp, ntiles(c1))
    Yt = _run_expert(Xt, wt, bt, ntiles(c2))
    Ym = _run_expert(Xm, wm, bm, ntiles(c3))

    y1 = jnp.take(Yp, jnp.where(is1 == 1, r1, 0), axis=0)
    y2 = jnp.take(Yt, jnp.where(is2 == 1, r2, 0), axis=0)
    y3 = jnp.take(Ym, jnp.where(is3 == 1, r3, 0), axis=0)
    wait = jnp.broadcast_to(wait_emb, (N, wait_emb.shape[0]))
    tc = t[:, None]
    return jnp.where(tc == 0, wait,
                     jnp.where(tc == 1, y1, jnp.where(tc == 2, y2, y3)))
